# baseline (speedup 1.0000x reference)
import jax
import jax.numpy as jnp
import numpy as np
from jax import lax
from jax.experimental import pallas as pl
from jax.experimental.pallas import tpu as pltpu
from jax.experimental.pallas import tpu_sc as plsc

_B, _L, _V, _E = 4096, 200, 32128, 128
_NC, _NS = 2, 16
_NW = _NC * _NS
_BPW = _B // _NW
_IPW = _BPW * _L
_NL = 16
_EV = _E // _NL
_EC = _E // 32
_C0 = 128
_C1 = _L - _C0
_G = 8
_NG = _L // _G

_PERM = np.empty((_E,), np.int32)
for _base in range(0, _E, 32):
    for _i in range(16):
        _PERM[_base + 2 * _i] = _base + _i
        _PERM[_base + 2 * _i + 1] = _base + 16 + _i


def _body(x_hbm, table_hbm, out_hbm, idx_v, rows0, rows1, out_stage,
          sem0, sem1):
    wid = lax.axis_index("s") * _NC + lax.axis_index("c")
    pltpu.sync_copy(x_hbm.at[pl.ds(wid * _IPW, _IPW)], idx_v)

    def start(b, rows, sem):
        off = pl.multiple_of(b * _L, 8)
        pltpu.async_copy(
            table_hbm.at[idx_v.at[pl.ds(off, _C0)]], rows.at[pl.ds(0, _C0)], sem)
        pltpu.async_copy(
            table_hbm.at[idx_v.at[pl.ds(off + _C0, _C1)]],
            rows.at[pl.ds(_C0, _C1)], sem)

    def wait(rows, sem):
        pltpu.make_async_copy(table_hbm.at[pl.ds(0, _L)], rows, sem).wait()

    def reduce_store(rows, b):
        zero = jnp.zeros((_NL,), jnp.float32)
        shift = jnp.int32(16)
        mask = jnp.int32(-65536)

        @plsc.parallel_loop(0, _L, unroll=2, carry=(zero,) * _EV)
        def acc(j, a):
            new = list(a)
            for k in range(_EC):
                w = rows[j, pl.ds(_NL * k, _NL)]
                lo = lax.bitcast_convert_type(w << shift, jnp.float32)
                hi = lax.bitcast_convert_type(w & mask, jnp.float32)
                new[2 * k] = new[2 * k] + lo
                new[2 * k + 1] = new[2 * k + 1] + hi
            return tuple(new)

        for k in range(_EV):
            out_stage[b, pl.ds(k * _NL, _NL)] = acc[k]

    start(0, rows0, sem0)
    pairs = _BPW // 2

    def pair(i, carry):
        b0 = 2 * i
        start(b0 + 1, rows1, sem1)
        wait(rows0, sem0)
        reduce_store(rows0, b0)

        @pl.when(i < pairs - 1)
        def _():
            start(b0 + 2, rows0, sem0)

        wait(rows1, sem1)
        reduce_store(rows1, b0 + 1)
        return carry

    lax.fori_loop(0, pairs, pair, 0)
    pltpu.sync_copy(out_stage, out_hbm.at[pl.ds(wid * _BPW, _BPW)])


def kernel(x, table):
    xf = x.reshape(-1)
    tb = table.astype(jnp.bfloat16)[:, _PERM]
    tb32 = jax.lax.bitcast_convert_type(
        tb.reshape(_V, _E // 2, 2), jnp.int32)
    mesh = plsc.VectorSubcoreMesh(core_axis_name="c", subcore_axis_name="s")
    f = pl.kernel(
        _body,
        out_type=jax.ShapeDtypeStruct((_B, _E), jnp.float32),
        mesh=mesh,
        compiler_params=pltpu.CompilerParams(use_tc_tiling_on_sc=False),
        scratch_types=[
            pltpu.VMEM((_IPW,), jnp.int32),
            pltpu.VMEM((_L, _E // 2), jnp.int32),
            pltpu.VMEM((_L, _E // 2), jnp.int32),
            pltpu.VMEM((_BPW, _E), jnp.float32),
            pltpu.SemaphoreType.DMA,
            pltpu.SemaphoreType.DMA,
        ],
    )
    return f(xf, tb32)


# speedup vs baseline: 1.0477x; 1.0477x over previous
import jax
import jax.numpy as jnp
import numpy as np
from jax import lax
from jax.experimental import pallas as pl
from jax.experimental.pallas import tpu as pltpu
from jax.experimental.pallas import tpu_sc as plsc

_B, _L, _V, _E = 4096, 200, 32128, 128
_NC, _NS = 2, 16
_NW = _NC * _NS
_BPW = _B // _NW
_IPW = _BPW * _L
_NL = 16
_EV = _E // _NL
_EC = _E // 32
_C0 = 128
_C1 = _L - _C0
_G = 8
_NG = _L // _G

_UNPERM = np.empty((_E,), np.int32)
for _c in range(_E):
    _k, _r = _c // 32, _c % 32
    _UNPERM[_c] = 32 * _k + (_r // 2) + 16 * (_r & 1)


def _body(x_hbm, table_hbm, out_hbm, idx_v, rows0, rows1, out_stage,
          sem0, sem1):
    wid = lax.axis_index("s") * _NC + lax.axis_index("c")
    pltpu.sync_copy(x_hbm.at[pl.ds(wid * _IPW, _IPW)], idx_v)

    def start(b, rows, sem):
        off = pl.multiple_of(b * _L, 8)
        pltpu.async_copy(
            table_hbm.at[idx_v.at[pl.ds(off, _C0)]], rows.at[pl.ds(0, _C0)], sem)
        pltpu.async_copy(
            table_hbm.at[idx_v.at[pl.ds(off + _C0, _C1)]],
            rows.at[pl.ds(_C0, _C1)], sem)

    def wait(rows, sem):
        pltpu.make_async_copy(table_hbm.at[pl.ds(0, _L)], rows, sem).wait()

    def reduce_store(rows, b):
        zero = jnp.zeros((_NL,), jnp.float32)
        shift = jnp.int32(16)
        mask = jnp.int32(-65536)

        @plsc.parallel_loop(0, _L, unroll=2, carry=(zero,) * _EV)
        def acc(j, a):
            new = list(a)
            for k in range(_EC):
                w = rows[j, pl.ds(_NL * k, _NL)]
                lo = lax.bitcast_convert_type(w << shift, jnp.float32)
                hi = lax.bitcast_convert_type(w & mask, jnp.float32)
                new[2 * k] = new[2 * k] + lo
                new[2 * k + 1] = new[2 * k + 1] + hi
            return tuple(new)

        base = pl.multiple_of(b * _E, 8)
        for m in range(_EV):
            out_stage[pl.ds(base + m * _NL, _NL)] = acc[m]

    start(0, rows0, sem0)
    pairs = _BPW // 2

    def pair(i, carry):
        b0 = 2 * i
        start(b0 + 1, rows1, sem1)
        wait(rows0, sem0)
        reduce_store(rows0, b0)

        @pl.when(i < pairs - 1)
        def _():
            start(b0 + 2, rows0, sem0)

        wait(rows1, sem1)
        reduce_store(rows1, b0 + 1)
        return carry

    lax.fori_loop(0, pairs, pair, 0)
    pltpu.sync_copy(out_stage,
                    out_hbm.at[pl.ds(wid * (_BPW * _E), _BPW * _E)])


def kernel(x, table):
    xf = x.reshape(-1)
    tb = table.astype(jnp.bfloat16)
    tb32 = jax.lax.bitcast_convert_type(
        tb.reshape(_V, _E // 2, 2), jnp.int32)
    mesh = plsc.VectorSubcoreMesh(core_axis_name="c", subcore_axis_name="s")
    f = pl.kernel(
        _body,
        out_type=jax.ShapeDtypeStruct((_B * _E,), jnp.float32),
        mesh=mesh,
        compiler_params=pltpu.CompilerParams(use_tc_tiling_on_sc=False),
        scratch_types=[
            pltpu.VMEM((_IPW,), jnp.int32),
            pltpu.VMEM((_L, _E // 2), jnp.int32),
            pltpu.VMEM((_L, _E // 2), jnp.int32),
            pltpu.VMEM((_BPW * _E,), jnp.float32),
            pltpu.SemaphoreType.DMA,
            pltpu.SemaphoreType.DMA,
        ],
    )
    return f(xf, tb32).reshape(_B, _E)[:, _UNPERM]
